# baseline (speedup 1.0000x reference)
import functools

import jax
import jax.numpy as jnp
from jax.experimental import pallas as pl
from jax.experimental.pallas import tpu as pltpu

N_FF = 5000
N_FFPE = 5000
N_PATH = N_FF + N_FFPE
E_PER = 53248
K_TOP = 16
G_TOTAL = 6


def _relu6(x):
    return jnp.minimum(jnp.maximum(x, 0.0), 6.0)


def _mlp2_kernel(x_ref, w1_ref, b1_ref, w2_ref, b2_ref, o_ref):
    h = _relu6(jnp.dot(x_ref[...], w1_ref[0], preferred_element_type=jnp.float32)
               + b1_ref[0, 0])
    o_ref[...] = _relu6(jnp.dot(h, w2_ref[0], preferred_element_type=jnp.float32)
                        + b2_ref[0, 0])


def _path_mlp(x_all, w1s, b1s, w2s, b2s):
    blk = 1000
    n_blk = N_PATH // blk
    half = n_blk // 2
    grid = (n_blk,)
    return pl.pallas_call(
        _mlp2_kernel,
        grid=grid,
        in_specs=[
            pl.BlockSpec((blk, 1024), lambda i: (i, 0)),
            pl.BlockSpec((1, 1024, 256), lambda i: (i // half, 0, 0)),
            pl.BlockSpec((1, 1, 256), lambda i: (i // half, 0, 0)),
            pl.BlockSpec((1, 256, 256), lambda i: (i // half, 0, 0)),
            pl.BlockSpec((1, 1, 256), lambda i: (i // half, 0, 0)),
        ],
        out_specs=pl.BlockSpec((blk, 256), lambda i: (i, 0)),
        out_shape=jax.ShapeDtypeStruct((N_PATH, 256), jnp.float32),
    )(x_all, w1s, b1s, w2s, b2s)


def _build_incidence(edge_ei):
    starts = jnp.asarray(edge_ei[0])
    ends = jnp.asarray(edge_ei[1])
    mask = jnp.zeros((N_PATH,), dtype=jnp.int32).at[starts].set(1)
    csum = jnp.cumsum(mask)
    n = csum[-1]
    inv = csum[starts] - 1
    rank_idx = jnp.where(mask == 1, csum - 1, N_PATH)
    uniq_pad = jnp.full((N_PATH,), N_PATH, dtype=starts.dtype).at[rank_idx].set(
        jnp.arange(N_PATH, dtype=starts.dtype), mode='drop')
    mem_v = jnp.concatenate([ends, uniq_pad])
    mem_e = jnp.concatenate([inv.astype(ends.dtype),
                             jnp.arange(N_PATH, dtype=ends.dtype)])
    valid = jnp.concatenate([jnp.ones((ends.shape[0],), dtype=bool),
                             jnp.arange(N_PATH) < n])
    big = N_PATH * N_PATH
    key = jnp.where(valid, mem_v * N_PATH + mem_e, big)
    skey = jnp.sort(key)
    is_val = skey < big
    first = jnp.concatenate([jnp.array([True]), skey[1:] != skey[:-1]])
    keep = is_val & first
    return skey // N_PATH, skey % N_PATH, keep, n


def _combined_incidence(ff_ei, ffpe_ei, share_ei):
    v1, e1, k1, n1 = _build_incidence(ff_ei)
    v2, e2, k2, n2 = _build_incidence(ffpe_ei)
    v3, e3, k3, n3 = _build_incidence(share_ei)
    pv = jnp.concatenate([v1, v2, v3])
    pe = jnp.concatenate([e1, e2 + n1, e3 + n1 + n2])
    keep = jnp.concatenate([k1, k2, k3])
    pv = jnp.where(keep, pv, N_PATH)
    pe = jnp.where(keep, pe, 3 * N_PATH)
    return pv, pe, 3 * N_PATH


def _v2v_mean(X, v_idx, e_idx, num_e, num_v):
    ones = jnp.ones((v_idx.shape[0],), dtype=X.dtype)
    e_deg = jnp.maximum(jax.ops.segment_sum(ones, e_idx, num_segments=num_e), 1.0)
    Xe = jax.ops.segment_sum(X[v_idx], e_idx, num_segments=num_e) / e_deg[:, None]
    v_deg = jnp.maximum(jax.ops.segment_sum(ones, v_idx, num_segments=num_v), 1.0)
    Xv = jax.ops.segment_sum(Xe[e_idx], v_idx, num_segments=num_v) / v_deg[:, None]
    return Xv


def _layer_norm(x, g, b, eps=1e-5):
    mu = jnp.mean(x, axis=-1, keepdims=True)
    var = jnp.var(x, axis=-1, keepdims=True)
    return (x - mu) / jnp.sqrt(var + eps) * g + b


def kernel(x_ff, x_ffpe, x_omic1, x_omic2, x_omic3, x_omic4, x_omic5, x_omic6,
           ff_edge_index, ffpe_edge_index, share_edge, params):
    p = params
    pv, pe, pne = _combined_incidence(ff_edge_index, ffpe_edge_index, share_edge)

    x_all = jnp.concatenate([x_ff[0], x_ffpe[0]], axis=0)
    w1s = jnp.stack([p['ff_w1'].T, p['fp_w1'].T])
    b1s = jnp.stack([p['ff_b1'], p['fp_b1']])[:, None, :]
    w2s = jnp.stack([p['ff_w2'].T, p['fp_w2'].T])
    b2s = jnp.stack([p['ff_b2'], p['fp_b2']])[:, None, :]
    path = _path_mlp(x_all, w1s, b1s, w2s, b2s)

    for j in range(3):
        path = path @ p['gp%d_w' % j].T + p['gp%d_b' % j]
        path = _v2v_mean(path, pv, pe, pne, N_PATH)
        path = jax.nn.relu(path)

    omics = [x_omic1, x_omic2, x_omic3, x_omic4, x_omic5, x_omic6]
    gens = []
    for i in range(6):
        g = jax.nn.elu(omics[i] @ p['g%d_w1' % i].T + p['g%d_b1' % i])
        g = jax.nn.elu(g @ p['g%d_w2' % i].T + p['g%d_b2' % i])
        gens.append(g)
    gen = jnp.stack(gens)

    attn = gen @ path.T
    _, top_idx = jax.lax.top_k(attn, K_TOP)
    token = jnp.concatenate([gen, path], axis=0)
    cv = jnp.concatenate([jnp.arange(G_TOTAL), (top_idx + G_TOTAL).reshape(-1)])
    ce = jnp.concatenate([jnp.arange(G_TOTAL),
                          jnp.repeat(jnp.arange(G_TOTAL), K_TOP)])
    for j in range(3):
        token = token @ p['gg%d_w' % j].T + p['gg%d_b' % j]
        token = _v2v_mean(token, cv, ce, G_TOTAL, N_PATH + G_TOTAL)
        token = jax.nn.relu(token)
    tc = token[None]
    tc = jax.nn.gelu(tc @ p['ffn_w1'].T + p['ffn_b1'],
                     approximate=False) @ p['ffn_w2'].T + p['ffn_b2']
    tc = _layer_norm(tc, p['ln_g'], p['ln_b'])
    gene_embed = jnp.mean(tc[:, :G_TOTAL, :], axis=1)
    path_embed = jnp.mean(tc[:, G_TOTAL:, :], axis=1)
    fusion = _relu6(jnp.concatenate([gene_embed, path_embed], axis=1)
                    @ p['mm_w'].T + p['mm_b'])
    logits = fusion @ p['cls_w'].T + p['cls_b']
    return (logits, path, gen)


# speedup vs baseline: 1.0183x; 1.0183x over previous
import functools

import jax
import jax.numpy as jnp
from jax.experimental import pallas as pl
from jax.experimental.pallas import tpu as pltpu

N_FF = 5000
N_FFPE = 5000
N_PATH = N_FF + N_FFPE
E_PER = 53248
K_TOP = 16
G_TOTAL = 6


def _relu6(x):
    return jnp.minimum(jnp.maximum(x, 0.0), 6.0)


def _mlp2_kernel(x_ref, w1_ref, b1_ref, w2_ref, b2_ref, o_ref):
    h = _relu6(jnp.dot(x_ref[...], w1_ref[0], preferred_element_type=jnp.float32)
               + b1_ref[0, 0])
    o_ref[...] = _relu6(jnp.dot(h, w2_ref[0], preferred_element_type=jnp.float32)
                        + b2_ref[0, 0])


def _path_mlp(x_all, w1s, b1s, w2s, b2s):
    blk = 1000
    n_blk = N_PATH // blk
    half = n_blk // 2
    grid = (n_blk,)
    return pl.pallas_call(
        _mlp2_kernel,
        grid=grid,
        in_specs=[
            pl.BlockSpec((blk, 1024), lambda i: (i, 0)),
            pl.BlockSpec((1, 1024, 256), lambda i: (i // half, 0, 0)),
            pl.BlockSpec((1, 1, 256), lambda i: (i // half, 0, 0)),
            pl.BlockSpec((1, 256, 256), lambda i: (i // half, 0, 0)),
            pl.BlockSpec((1, 1, 256), lambda i: (i // half, 0, 0)),
        ],
        out_specs=pl.BlockSpec((blk, 256), lambda i: (i, 0)),
        out_shape=jax.ShapeDtypeStruct((N_PATH, 256), jnp.float32),
    )(x_all, w1s, b1s, w2s, b2s)


def _build_incidence(edge_ei):
    starts = jnp.asarray(edge_ei[0])
    ends = jnp.asarray(edge_ei[1])
    mask = jnp.zeros((N_PATH,), dtype=jnp.int32).at[starts].set(1)
    csum = jnp.cumsum(mask)
    n = csum[-1]
    inv = csum[starts] - 1
    rank_idx = jnp.where(mask == 1, csum - 1, N_PATH)
    uniq_pad = jnp.full((N_PATH,), N_PATH, dtype=starts.dtype).at[rank_idx].set(
        jnp.arange(N_PATH, dtype=starts.dtype), mode='drop')
    mem_v = jnp.concatenate([ends, uniq_pad])
    mem_e = jnp.concatenate([inv.astype(ends.dtype),
                             jnp.arange(N_PATH, dtype=ends.dtype)])
    valid = jnp.concatenate([jnp.ones((ends.shape[0],), dtype=bool),
                             jnp.arange(N_PATH) < n])
    big = N_PATH * N_PATH
    key = jnp.where(valid, mem_v * N_PATH + mem_e, big)
    skey = jnp.sort(key)
    is_val = skey < big
    first = jnp.concatenate([jnp.array([True]), skey[1:] != skey[:-1]])
    keep = is_val & first
    return skey // N_PATH, skey % N_PATH, keep, n


def _combined_incidence(ff_ei, ffpe_ei, share_ei):
    v1, e1, k1, n1 = _build_incidence(ff_ei)
    v2, e2, k2, n2 = _build_incidence(ffpe_ei)
    v3, e3, k3, n3 = _build_incidence(share_ei)
    pv = jnp.concatenate([v1, v2, v3])
    pe = jnp.concatenate([e1, e2 + n1, e3 + n1 + n2])
    keep = jnp.concatenate([k1, k2, k3])
    pv = jnp.where(keep, pv, N_PATH)
    pe = jnp.where(keep, pe, 3 * N_PATH)
    return pv, pe, 3 * N_PATH


def _v2v_mean(X, v_idx, e_idx, num_e, num_v):
    ones = jnp.ones((v_idx.shape[0],), dtype=X.dtype)
    e_deg = jnp.maximum(jax.ops.segment_sum(ones, e_idx, num_segments=num_e), 1.0)
    Xe = jax.ops.segment_sum(X[v_idx], e_idx, num_segments=num_e) / e_deg[:, None]
    v_deg = jnp.maximum(jax.ops.segment_sum(ones, v_idx, num_segments=num_v), 1.0)
    Xv = jax.ops.segment_sum(Xe[e_idx], v_idx, num_segments=num_v) / v_deg[:, None]
    return Xv


def _layer_norm(x, g, b, eps=1e-5):
    mu = jnp.mean(x, axis=-1, keepdims=True)
    var = jnp.var(x, axis=-1, keepdims=True)
    return (x - mu) / jnp.sqrt(var + eps) * g + b


def _topk_kernel(gen_ref, path_ref, o_ref):
    attn = jnp.dot(gen_ref[...], path_ref[...].T,
                   preferred_element_type=jnp.float32)
    idxmat = jax.lax.broadcasted_iota(jnp.int32, (8, N_PATH), 1)
    for k in range(K_TOP):
        m = jnp.max(attn, axis=1, keepdims=True)
        cand = jnp.where(attn == m, idxmat, jnp.int32(2**30))
        idx = jnp.min(cand, axis=1, keepdims=True)
        o_ref[:, pl.ds(k, 1)] = idx
        attn = jnp.where(idxmat == idx, -jnp.inf, attn)


def _topk(gen, path):
    gen_pad = jnp.zeros((8, 256), jnp.float32).at[:G_TOTAL].set(gen)
    out = pl.pallas_call(
        _topk_kernel,
        out_shape=jax.ShapeDtypeStruct((8, K_TOP), jnp.int32),
    )(gen_pad, path)
    return out[:G_TOTAL]


def _token_kernel(gen_ref, path_ref, cv_ref, ce_ref, ti_ref,
                  ggw_ref, ggb_ref, w1_ref, b1_ref, w2_ref, b2_ref,
                  lng_ref, lnb_ref, mmw_ref, mmb_ref, clsw_ref, clsb_ref,
                  o_ref, t_ref):
    t_ref[...] = jnp.zeros((128, 256), jnp.float32)
    t_ref[0:8, :] = gen_ref[...]

    def body(k, _):
        idx = ti_ref[k // K_TOP, k % K_TOP]
        t_ref[pl.ds(k + G_TOTAL, 1), :] = path_ref[pl.ds(idx, 1), :]
        return 0

    jax.lax.fori_loop(0, 96, body, 0)

    cv = cv_ref[...]
    ce = ce_ref[...]
    C = (cv == cv.T).astype(jnp.float32)
    deg = jnp.sum(C, axis=1, keepdims=True)
    e_row = jax.lax.broadcasted_iota(jnp.int32, (8, 128), 0)
    E_mat = jnp.where(ce == e_row, 1.0 / 17.0, 0.0)
    e_col = jax.lax.broadcasted_iota(jnp.int32, (128, 8), 1)
    oh_ke = (ce.T == e_col).astype(jnp.float32)
    G_mat = jnp.dot(C, oh_ke, preferred_element_type=jnp.float32) / deg

    T = t_ref[...]
    for j in range(3):
        T_lin = jnp.dot(T, ggw_ref[j], preferred_element_type=jnp.float32) \
            + ggb_ref[j][None, :]
        Xe = jnp.dot(E_mat, T_lin, preferred_element_type=jnp.float32)
        T = jnp.maximum(jnp.dot(G_mat, Xe, preferred_element_type=jnp.float32),
                        0.0)

    w1 = w1_ref[...]
    b1 = b1_ref[...]
    w2 = w2_ref[...]
    b2 = b2_ref[...]
    lng = lng_ref[...]
    lnb = lnb_ref[...]

    def gelu_exact(z):
        return 0.5 * z * (1.0 + jax.lax.erf(z * (2.0 ** -0.5)))

    def ffn_ln(x):
        h = gelu_exact(jnp.dot(x, w1, preferred_element_type=jnp.float32)
                       + b1)
        f = jnp.dot(h, w2, preferred_element_type=jnp.float32) + b2
        mu = jnp.mean(f, axis=-1, keepdims=True)
        var = jnp.mean((f - mu) ** 2, axis=-1, keepdims=True)
        return (f - mu) / jnp.sqrt(var + 1e-5) * lng + lnb

    F = ffn_ln(T)
    tc_zero = ffn_ln(jnp.zeros((8, 256), jnp.float32))[0:1]

    r_col = jax.lax.broadcasted_iota(jnp.int32, (1, 128), 1)
    m_g = jnp.where(r_col < G_TOTAL, 1.0 / G_TOTAL, 0.0)
    gene_embed = jnp.dot(m_g, F, preferred_element_type=jnp.float32)

    row_i = jax.lax.broadcasted_iota(jnp.int32, (128, 128), 0)
    col_i = jax.lax.broadcasted_iota(jnp.int32, (128, 128), 1)
    L = jnp.where(col_i < row_i, 1.0, 0.0)
    S = jnp.sum(C * L, axis=1, keepdims=True)
    r_row = jax.lax.broadcasted_iota(jnp.int32, (128, 1), 0)
    m_path = jnp.where((S == 0.0) & (r_row >= G_TOTAL) & (r_row < 102),
                       1.0, 0.0)
    n_unique = jnp.sum(m_path)
    path_sum = jnp.dot(m_path.T, F, preferred_element_type=jnp.float32)
    path_embed = (path_sum + (N_PATH - n_unique) * tc_zero) / N_PATH

    fused = jnp.concatenate([gene_embed, path_embed], axis=1)
    fusion = _relu6(jnp.dot(fused, mmw_ref[...],
                            preferred_element_type=jnp.float32) + mmb_ref[...])
    o_ref[...] = jnp.dot(fusion, clsw_ref[...],
                         preferred_element_type=jnp.float32) + clsb_ref[...]


def _token_head(gen, path, top_idx, p):
    cv = jnp.concatenate([
        jnp.arange(G_TOTAL, dtype=jnp.int32),
        top_idx.reshape(-1).astype(jnp.int32) + G_TOTAL,
        jnp.arange(102, 128, dtype=jnp.int32) + 100000,
    ])[:, None]
    ce = jnp.concatenate([
        jnp.arange(G_TOTAL, dtype=jnp.int32),
        jnp.repeat(jnp.arange(G_TOTAL, dtype=jnp.int32), K_TOP),
        jnp.full((26,), G_TOTAL + 1, jnp.int32),
    ])[None, :]
    gen_pad = jnp.zeros((8, 256), jnp.float32).at[:G_TOTAL].set(gen)
    ggw = jnp.stack([p['gg%d_w' % j].T for j in range(3)])
    ggb = jnp.stack([p['gg%d_b' % j] for j in range(3)])
    return pl.pallas_call(
        _token_kernel,
        in_specs=[
            pl.BlockSpec((8, 256), lambda: (0, 0)),
            pl.BlockSpec((N_PATH, 256), lambda: (0, 0)),
            pl.BlockSpec((128, 1), lambda: (0, 0)),
            pl.BlockSpec((1, 128), lambda: (0, 0)),
            pl.BlockSpec(memory_space=pltpu.SMEM),
            pl.BlockSpec((3, 256, 256), lambda: (0, 0, 0)),
            pl.BlockSpec((3, 256), lambda: (0, 0)),
            pl.BlockSpec((256, 512), lambda: (0, 0)),
            pl.BlockSpec((1, 512), lambda: (0, 0)),
            pl.BlockSpec((512, 256), lambda: (0, 0)),
            pl.BlockSpec((1, 256), lambda: (0, 0)),
            pl.BlockSpec((1, 256), lambda: (0, 0)),
            pl.BlockSpec((1, 256), lambda: (0, 0)),
            pl.BlockSpec((512, 128), lambda: (0, 0)),
            pl.BlockSpec((1, 128), lambda: (0, 0)),
            pl.BlockSpec((128, 4), lambda: (0, 0)),
            pl.BlockSpec((1, 4), lambda: (0, 0)),
        ],
        out_specs=pl.BlockSpec((1, 4), lambda: (0, 0)),
        out_shape=jax.ShapeDtypeStruct((1, 4), jnp.float32),
        scratch_shapes=[pltpu.VMEM((128, 256), jnp.float32)],
    )(gen_pad, path, cv, ce, top_idx.astype(jnp.int32),
      ggw, ggb, p['ffn_w1'].T, p['ffn_b1'][None], p['ffn_w2'].T,
      p['ffn_b2'][None], p['ln_g'][None], p['ln_b'][None],
      p['mm_w'].T, p['mm_b'][None], p['cls_w'].T, p['cls_b'][None])


def kernel(x_ff, x_ffpe, x_omic1, x_omic2, x_omic3, x_omic4, x_omic5, x_omic6,
           ff_edge_index, ffpe_edge_index, share_edge, params):
    p = params
    pv, pe, pne = _combined_incidence(ff_edge_index, ffpe_edge_index, share_edge)

    x_all = jnp.concatenate([x_ff[0], x_ffpe[0]], axis=0)
    w1s = jnp.stack([p['ff_w1'].T, p['fp_w1'].T])
    b1s = jnp.stack([p['ff_b1'], p['fp_b1']])[:, None, :]
    w2s = jnp.stack([p['ff_w2'].T, p['fp_w2'].T])
    b2s = jnp.stack([p['ff_b2'], p['fp_b2']])[:, None, :]
    path = _path_mlp(x_all, w1s, b1s, w2s, b2s)

    for j in range(3):
        path = path @ p['gp%d_w' % j].T + p['gp%d_b' % j]
        path = _v2v_mean(path, pv, pe, pne, N_PATH)
        path = jax.nn.relu(path)

    omics = [x_omic1, x_omic2, x_omic3, x_omic4, x_omic5, x_omic6]
    gens = []
    for i in range(6):
        g = jax.nn.elu(omics[i] @ p['g%d_w1' % i].T + p['g%d_b1' % i])
        g = jax.nn.elu(g @ p['g%d_w2' % i].T + p['g%d_b2' % i])
        gens.append(g)
    gen = jnp.stack(gens)

    top_idx = _topk(gen, path)
    logits = _token_head(gen, path, top_idx, p)
    return (logits, path, gen)


# speedup vs baseline: 1.8097x; 1.7771x over previous
import functools

import jax
import jax.numpy as jnp
from jax import lax
from jax.experimental import pallas as pl
from jax.experimental.pallas import tpu as pltpu
from jax.experimental.pallas import tpu_sc as plsc

N_FF = 5000
N_FFPE = 5000
N_PATH = N_FF + N_FFPE
E_PER = 53248
K_TOP = 16
G_TOTAL = 6


def _relu6(x):
    return jnp.minimum(jnp.maximum(x, 0.0), 6.0)


def _mlp2_kernel(x_ref, w1_ref, b1_ref, w2_ref, b2_ref, o_ref):
    h = _relu6(jnp.dot(x_ref[...], w1_ref[0], preferred_element_type=jnp.float32)
               + b1_ref[0, 0])
    o_ref[...] = _relu6(jnp.dot(h, w2_ref[0], preferred_element_type=jnp.float32)
                        + b2_ref[0, 0])


def _path_mlp(x_all, w1s, b1s, w2s, b2s):
    blk = 1000
    n_blk = N_PATH // blk
    half = n_blk // 2
    grid = (n_blk,)
    return pl.pallas_call(
        _mlp2_kernel,
        grid=grid,
        in_specs=[
            pl.BlockSpec((blk, 1024), lambda i: (i, 0)),
            pl.BlockSpec((1, 1024, 256), lambda i: (i // half, 0, 0)),
            pl.BlockSpec((1, 1, 256), lambda i: (i // half, 0, 0)),
            pl.BlockSpec((1, 256, 256), lambda i: (i // half, 0, 0)),
            pl.BlockSpec((1, 1, 256), lambda i: (i // half, 0, 0)),
        ],
        out_specs=pl.BlockSpec((blk, 256), lambda i: (i, 0)),
        out_shape=jax.ShapeDtypeStruct((N_PATH, 256), jnp.float32),
    )(x_all, w1s, b1s, w2s, b2s)


_BIG = 1 << 30
_NPAIR = 3 * (E_PER + N_PATH)
_KLEN = 190208
_EWIN = 240
_NEW = 128
_EROWS = _NEW * _EWIN
_VWIN = 160
_NVW = 64
_VROWS = _NVW * _VWIN
_CH = 128


def _pair_keys(ff_ei, ffpe_ei, share_ei):
    pvs, pes, vals = [], [], []
    eoff = jnp.int32(0)
    for ei in (ff_ei, ffpe_ei, share_ei):
        starts = jnp.asarray(ei[0]).astype(jnp.int32)
        ends = jnp.asarray(ei[1]).astype(jnp.int32)
        mask = jnp.zeros((N_PATH,), jnp.int32).at[starts].set(1)
        csum = jnp.cumsum(mask)
        n = csum[-1]
        inv = csum[starts] - 1
        rank_idx = jnp.where(mask == 1, csum - 1, N_PATH)
        uniq = jnp.zeros((N_PATH,), jnp.int32).at[rank_idx].set(
            jnp.arange(N_PATH, dtype=jnp.int32), mode='drop')
        pvs += [ends, uniq]
        pes += [inv + eoff, jnp.arange(N_PATH, dtype=jnp.int32) + eoff]
        vals += [jnp.ones((E_PER,), bool), jnp.arange(N_PATH) < n]
        eoff = eoff + n
    pv = jnp.concatenate(pvs)
    pe = jnp.concatenate(pes)
    val = jnp.concatenate(vals)
    pad = jnp.full((_KLEN - _NPAIR,), _BIG, jnp.int32)
    key_e = jnp.concatenate([jnp.where(val, pe * 16384 + pv, _BIG), pad])
    key_v = jnp.concatenate([jnp.where(val, pv * 32768 + pe, _BIG), pad])
    ks_e = jnp.sort(key_e)
    ks_v = jnp.sort(key_v)
    b_e = jnp.searchsorted(
        ks_e, jnp.arange(_NEW + 1, dtype=jnp.int32) * (_EWIN * 16384),
    ).astype(jnp.int32)
    b_v = jnp.searchsorted(
        ks_v, jnp.arange(_NVW + 1, dtype=jnp.int32) * (_VWIN * 32768),
    ).astype(jnp.int32)
    dup_e = jnp.concatenate([jnp.array([False]), ks_e[1:] == ks_e[:-1]])
    dup_v = jnp.concatenate([jnp.array([False]), ks_v[1:] == ks_v[:-1]])
    ks_e = jnp.where(dup_e, _BIG, ks_e)
    ks_v = jnp.where(dup_v, _BIG, ks_v)
    b_e = jnp.concatenate([b_e, jnp.zeros((144 - _NEW - 1,), jnp.int32)])
    b_v = jnp.concatenate([b_v, jnp.zeros((80 - _NVW - 1,), jnp.int32)])
    ok_e = (ks_e < _BIG).astype(jnp.float32)
    ok_v = (ks_v < _BIG).astype(jnp.float32)
    deg_e = jnp.maximum(jax.ops.segment_sum(
        ok_e, jnp.minimum(ks_e >> 14, _EROWS - 1), num_segments=_EROWS), 1.0)
    deg_v = jnp.maximum(jax.ops.segment_sum(
        ok_v, jnp.minimum(ks_v >> 15, _VROWS - 1), num_segments=_VROWS), 1.0)
    return ks_e, ks_v, b_e, b_v, deg_e, deg_v


def _sel(vec_ref, i):
    return vec_ref[pl.ds(i, 16)][0]


_SC_MESH = dict(core_axis_name="c", subcore_axis_name="s")


def _seg_mean_kernel(nwin_per, win_rows, shift, vmask, relu):
    def body(key_hbm, b_hbm, table_hbm, deg_hbm, out_hbm,
             bvec, kb, gi, si, rows, dbuf, acc, sem):
        z16 = jnp.zeros((16,), jnp.float32)
        c = lax.axis_index("c")
        s = lax.axis_index("s")
        g = c * 16 + s
        pltpu.sync_copy(b_hbm, bvec)

        for wi in range(nwin_per):
            w = g * nwin_per + wi
            base = w * win_rows
            lo = _sel(bvec, w)
            hi = _sel(bvec, w + 1)
            lo_key = base << shift
            hi_key = (base + win_rows) << shift

            def zero_row(r, _):
                for i in range(16):
                    acc[r, pl.ds(i * 16, 16)] = z16
                return 0

            lax.fori_loop(0, win_rows + 8, zero_row, 0)

            astart = jnp.bitwise_and(lo, jnp.int32(-8))
            nch = (hi - astart + (_CH - 1)) // _CH

            def chunk(n, _):
                cstart = pl.multiple_of(astart + n * _CH, 8)
                pltpu.sync_copy(key_hbm.at[pl.ds(cstart, _CH)], kb)
                for i in range(8):
                    sl = pl.ds(i * 16, 16)
                    kk = kb[sl]
                    m = (kk >= lo_key) & (kk < hi_key)
                    gi[sl] = jnp.where(m, jnp.bitwise_and(kk, vmask), 0)
                    si[sl] = jnp.where(m, (kk >> shift) - base,
                                       jnp.int32(win_rows))
                pltpu.async_copy(table_hbm.at[gi], rows, sem).wait()

                def pair(p, _):
                    ep = si[pl.ds(p, 16)][0]
                    for cc in range(16):
                        sl = pl.ds(cc * 16, 16)
                        acc[ep, sl] = acc[ep, sl] + rows[p, sl]
                    return 0

                lax.fori_loop(0, _CH, pair, 0)
                return 0

            lax.fori_loop(0, jnp.maximum(nch, 0), chunk, 0)
            pltpu.sync_copy(deg_hbm.at[pl.ds(base, win_rows)],
                            dbuf.at[pl.ds(0, win_rows)])

            def wb(j, _):
                d = dbuf[pl.ds(j, 16)][0]
                for i in range(16):
                    sl = pl.ds(i * 16, 16)
                    v = acc[j, sl] / d
                    if relu:
                        v = jnp.maximum(v, 0.0)
                    acc[j, sl] = v
                return 0

            lax.fori_loop(0, win_rows, wb, 0)
            pltpu.sync_copy(acc.at[pl.ds(0, win_rows)],
                            out_hbm.at[pl.ds(base, win_rows)])

    return body


def _seg_mean(key_s, bounds, table, deg, nwin_per, win_rows, nrows_out,
              shift, vmask, relu, blen):
    kfn = _seg_mean_kernel(nwin_per, win_rows, shift, vmask, relu)

    @functools.partial(
        pl.kernel, mesh=plsc.VectorSubcoreMesh(**_SC_MESH),
        out_type=jax.ShapeDtypeStruct((nrows_out, 256), jnp.float32),
        scratch_types=[
            pltpu.VMEM((blen,), jnp.int32),
            pltpu.VMEM((_CH,), jnp.int32),
            pltpu.VMEM((_CH,), jnp.int32),
            pltpu.VMEM((_CH + 16,), jnp.int32),
            pltpu.VMEM((_CH, 256), jnp.float32),
            pltpu.VMEM((win_rows + 16,), jnp.float32),
            pltpu.VMEM((win_rows + 8, 256), jnp.float32),
            pltpu.SemaphoreType.DMA,
        ],
    )
    def k(key_hbm, b_hbm, table_hbm, deg_hbm, out_hbm,
          bvec, kb, gi, si, rows, dbuf, acc, sem):
        kfn(key_hbm, b_hbm, table_hbm, deg_hbm, out_hbm,
            bvec, kb, gi, si, rows, dbuf, acc, sem)

    return k(key_s, bounds, table, deg)


def _hop1(x, ks_e, b_e, deg_e):
    return _seg_mean(ks_e, b_e, x, deg_e, 4, _EWIN, _EROWS, 14,
                     jnp.int32(16383), False, 144)


def _hop2(xe, ks_v, b_v, deg_v):
    return _seg_mean(ks_v, b_v, xe, deg_v, 2, _VWIN, _VROWS, 15,
                     jnp.int32(32767), True, 80)


def _linear_kernel(x_ref, w_ref, b_ref, o_ref):
    o_ref[...] = jnp.dot(x_ref[...], w_ref[...],
                         preferred_element_type=jnp.float32) + b_ref[...]


def _linear(x, w_t, b):
    blk = 2000
    return pl.pallas_call(
        _linear_kernel,
        grid=(N_PATH // blk,),
        in_specs=[
            pl.BlockSpec((blk, 256), lambda i: (i, 0)),
            pl.BlockSpec((256, 256), lambda i: (0, 0)),
            pl.BlockSpec((1, 256), lambda i: (0, 0)),
        ],
        out_specs=pl.BlockSpec((blk, 256), lambda i: (i, 0)),
        out_shape=jax.ShapeDtypeStruct((N_PATH, 256), jnp.float32),
    )(x, w_t, b[None])


def _topk_kernel(gen_ref, path_ref, o_ref):
    attn = jnp.dot(gen_ref[...], path_ref[...].T,
                   preferred_element_type=jnp.float32)
    idxmat = jax.lax.broadcasted_iota(jnp.int32, (8, N_PATH), 1)
    for k in range(K_TOP):
        m = jnp.max(attn, axis=1, keepdims=True)
        cand = jnp.where(attn == m, idxmat, jnp.int32(2**30))
        idx = jnp.min(cand, axis=1, keepdims=True)
        o_ref[:, pl.ds(k, 1)] = idx
        attn = jnp.where(idxmat == idx, -jnp.inf, attn)


def _topk(gen, path):
    gen_pad = jnp.zeros((8, 256), jnp.float32).at[:G_TOTAL].set(gen)
    out = pl.pallas_call(
        _topk_kernel,
        out_shape=jax.ShapeDtypeStruct((8, K_TOP), jnp.int32),
    )(gen_pad, path)
    return out[:G_TOTAL]


def _token_kernel(gen_ref, path_ref, cv_ref, ce_ref, ti_ref,
                  ggw_ref, ggb_ref, w1_ref, b1_ref, w2_ref, b2_ref,
                  lng_ref, lnb_ref, mmw_ref, mmb_ref, clsw_ref, clsb_ref,
                  o_ref, t_ref):
    t_ref[...] = jnp.zeros((128, 256), jnp.float32)
    t_ref[0:8, :] = gen_ref[...]

    def body(k, _):
        idx = ti_ref[k // K_TOP, k % K_TOP]
        t_ref[pl.ds(k + G_TOTAL, 1), :] = path_ref[pl.ds(idx, 1), :]
        return 0

    jax.lax.fori_loop(0, 96, body, 0)

    cv = cv_ref[...]
    ce = ce_ref[...]
    C = (cv == cv.T).astype(jnp.float32)
    deg = jnp.sum(C, axis=1, keepdims=True)
    e_row = jax.lax.broadcasted_iota(jnp.int32, (8, 128), 0)
    E_mat = jnp.where(ce == e_row, 1.0 / 17.0, 0.0)
    e_col = jax.lax.broadcasted_iota(jnp.int32, (128, 8), 1)
    oh_ke = (ce.T == e_col).astype(jnp.float32)
    G_mat = jnp.dot(C, oh_ke, preferred_element_type=jnp.float32) / deg

    T = t_ref[...]
    for j in range(3):
        T_lin = jnp.dot(T, ggw_ref[j], preferred_element_type=jnp.float32) \
            + ggb_ref[j][None, :]
        Xe = jnp.dot(E_mat, T_lin, preferred_element_type=jnp.float32)
        T = jnp.maximum(jnp.dot(G_mat, Xe, preferred_element_type=jnp.float32),
                        0.0)

    w1 = w1_ref[...]
    b1 = b1_ref[...]
    w2 = w2_ref[...]
    b2 = b2_ref[...]
    lng = lng_ref[...]
    lnb = lnb_ref[...]

    def gelu_exact(z):
        return 0.5 * z * (1.0 + jax.lax.erf(z * (2.0 ** -0.5)))

    def ffn_ln(x):
        h = gelu_exact(jnp.dot(x, w1, preferred_element_type=jnp.float32)
                       + b1)
        f = jnp.dot(h, w2, preferred_element_type=jnp.float32) + b2
        mu = jnp.mean(f, axis=-1, keepdims=True)
        var = jnp.mean((f - mu) ** 2, axis=-1, keepdims=True)
        return (f - mu) / jnp.sqrt(var + 1e-5) * lng + lnb

    F = ffn_ln(T)
    tc_zero = ffn_ln(jnp.zeros((8, 256), jnp.float32))[0:1]

    r_col = jax.lax.broadcasted_iota(jnp.int32, (1, 128), 1)
    m_g = jnp.where(r_col < G_TOTAL, 1.0 / G_TOTAL, 0.0)
    gene_embed = jnp.dot(m_g, F, preferred_element_type=jnp.float32)

    row_i = jax.lax.broadcasted_iota(jnp.int32, (128, 128), 0)
    col_i = jax.lax.broadcasted_iota(jnp.int32, (128, 128), 1)
    L = jnp.where(col_i < row_i, 1.0, 0.0)
    S = jnp.sum(C * L, axis=1, keepdims=True)
    r_row = jax.lax.broadcasted_iota(jnp.int32, (128, 1), 0)
    m_path = jnp.where((S == 0.0) & (r_row >= G_TOTAL) & (r_row < 102),
                       1.0, 0.0)
    n_unique = jnp.sum(m_path)
    path_sum = jnp.dot(m_path.T, F, preferred_element_type=jnp.float32)
    path_embed = (path_sum + (N_PATH - n_unique) * tc_zero) / N_PATH

    fused = jnp.concatenate([gene_embed, path_embed], axis=1)
    fusion = _relu6(jnp.dot(fused, mmw_ref[...],
                            preferred_element_type=jnp.float32) + mmb_ref[...])
    o_ref[...] = jnp.dot(fusion, clsw_ref[...],
                         preferred_element_type=jnp.float32) + clsb_ref[...]


def _token_head(gen, path, top_idx, p):
    cv = jnp.concatenate([
        jnp.arange(G_TOTAL, dtype=jnp.int32),
        top_idx.reshape(-1).astype(jnp.int32) + G_TOTAL,
        jnp.arange(102, 128, dtype=jnp.int32) + 100000,
    ])[:, None]
    ce = jnp.concatenate([
        jnp.arange(G_TOTAL, dtype=jnp.int32),
        jnp.repeat(jnp.arange(G_TOTAL, dtype=jnp.int32), K_TOP),
        jnp.full((26,), G_TOTAL + 1, jnp.int32),
    ])[None, :]
    gen_pad = jnp.zeros((8, 256), jnp.float32).at[:G_TOTAL].set(gen)
    ggw = jnp.stack([p['gg%d_w' % j].T for j in range(3)])
    ggb = jnp.stack([p['gg%d_b' % j] for j in range(3)])
    return pl.pallas_call(
        _token_kernel,
        in_specs=[
            pl.BlockSpec((8, 256), lambda: (0, 0)),
            pl.BlockSpec((N_PATH, 256), lambda: (0, 0)),
            pl.BlockSpec((128, 1), lambda: (0, 0)),
            pl.BlockSpec((1, 128), lambda: (0, 0)),
            pl.BlockSpec(memory_space=pltpu.SMEM),
            pl.BlockSpec((3, 256, 256), lambda: (0, 0, 0)),
            pl.BlockSpec((3, 256), lambda: (0, 0)),
            pl.BlockSpec((256, 512), lambda: (0, 0)),
            pl.BlockSpec((1, 512), lambda: (0, 0)),
            pl.BlockSpec((512, 256), lambda: (0, 0)),
            pl.BlockSpec((1, 256), lambda: (0, 0)),
            pl.BlockSpec((1, 256), lambda: (0, 0)),
            pl.BlockSpec((1, 256), lambda: (0, 0)),
            pl.BlockSpec((512, 128), lambda: (0, 0)),
            pl.BlockSpec((1, 128), lambda: (0, 0)),
            pl.BlockSpec((128, 4), lambda: (0, 0)),
            pl.BlockSpec((1, 4), lambda: (0, 0)),
        ],
        out_specs=pl.BlockSpec((1, 4), lambda: (0, 0)),
        out_shape=jax.ShapeDtypeStruct((1, 4), jnp.float32),
        scratch_shapes=[pltpu.VMEM((128, 256), jnp.float32)],
    )(gen_pad, path, cv, ce, top_idx.astype(jnp.int32),
      ggw, ggb, p['ffn_w1'].T, p['ffn_b1'][None], p['ffn_w2'].T,
      p['ffn_b2'][None], p['ln_g'][None], p['ln_b'][None],
      p['mm_w'].T, p['mm_b'][None], p['cls_w'].T, p['cls_b'][None])


def kernel(x_ff, x_ffpe, x_omic1, x_omic2, x_omic3, x_omic4, x_omic5, x_omic6,
           ff_edge_index, ffpe_edge_index, share_edge, params):
    p = params
    (ks_e, ks_v, b_e, b_v, deg_e, deg_v) = _pair_keys(
        ff_edge_index, ffpe_edge_index, share_edge)

    x_all = jnp.concatenate([x_ff[0], x_ffpe[0]], axis=0)
    w1s = jnp.stack([p['ff_w1'].T, p['fp_w1'].T])
    b1s = jnp.stack([p['ff_b1'], p['fp_b1']])[:, None, :]
    w2s = jnp.stack([p['ff_w2'].T, p['fp_w2'].T])
    b2s = jnp.stack([p['ff_b2'], p['fp_b2']])[:, None, :]
    path = _path_mlp(x_all, w1s, b1s, w2s, b2s)

    for j in range(3):
        xlin = _linear(path, p['gp%d_w' % j].T, p['gp%d_b' % j])
        xe = _hop1(xlin, ks_e, b_e, deg_e)
        xv = _hop2(xe, ks_v, b_v, deg_v)
        path = xv[:N_PATH]

    omics = [x_omic1, x_omic2, x_omic3, x_omic4, x_omic5, x_omic6]
    gens = []
    for i in range(6):
        g = jax.nn.elu(omics[i] @ p['g%d_w1' % i].T + p['g%d_b1' % i])
        g = jax.nn.elu(g @ p['g%d_w2' % i].T + p['g%d_b2' % i])
        gens.append(g)
    gen = jnp.stack(gens)

    top_idx = _topk(gen, path)
    logits = _token_head(gen, path, top_idx, p)
    return (logits, path, gen)


# speedup vs baseline: 2.4272x; 1.3413x over previous
import functools

import jax
import jax.numpy as jnp
from jax import lax
from jax.experimental import pallas as pl
from jax.experimental.pallas import tpu as pltpu
from jax.experimental.pallas import tpu_sc as plsc

N_FF = 5000
N_FFPE = 5000
N_PATH = N_FF + N_FFPE
E_PER = 53248
K_TOP = 16
G_TOTAL = 6


def _relu6(x):
    return jnp.minimum(jnp.maximum(x, 0.0), 6.0)


def _mlp2_kernel(x_ref, w1_ref, b1_ref, w2_ref, b2_ref, o_ref):
    h = _relu6(jnp.dot(x_ref[...], w1_ref[0], preferred_element_type=jnp.float32)
               + b1_ref[0, 0])
    o_ref[...] = _relu6(jnp.dot(h, w2_ref[0], preferred_element_type=jnp.float32)
                        + b2_ref[0, 0])


def _path_mlp(x_all, w1s, b1s, w2s, b2s):
    blk = 1000
    n_blk = N_PATH // blk
    half = n_blk // 2
    grid = (n_blk,)
    return pl.pallas_call(
        _mlp2_kernel,
        grid=grid,
        in_specs=[
            pl.BlockSpec((blk, 1024), lambda i: (i, 0)),
            pl.BlockSpec((1, 1024, 256), lambda i: (i // half, 0, 0)),
            pl.BlockSpec((1, 1, 256), lambda i: (i // half, 0, 0)),
            pl.BlockSpec((1, 256, 256), lambda i: (i // half, 0, 0)),
            pl.BlockSpec((1, 1, 256), lambda i: (i // half, 0, 0)),
        ],
        out_specs=pl.BlockSpec((blk, 256), lambda i: (i, 0)),
        out_shape=jax.ShapeDtypeStruct((N_PATH, 256), jnp.float32),
    )(x_all, w1s, b1s, w2s, b2s)


_BIG = 1 << 30
_NPAIR = 3 * (E_PER + N_PATH)
_KLEN = 190208
_EWIN = 240
_NEW = 128
_EROWS = _NEW * _EWIN
_VWIN = 160
_NVW = 64
_VROWS = _NVW * _VWIN
_CH = 128


def _pair_keys(ff_ei, ffpe_ei, share_ei):
    pvs, pes, vals = [], [], []
    eoff = jnp.int32(0)
    for ei in (ff_ei, ffpe_ei, share_ei):
        starts = jnp.asarray(ei[0]).astype(jnp.int32)
        ends = jnp.asarray(ei[1]).astype(jnp.int32)
        mask = jnp.zeros((N_PATH,), jnp.int32).at[starts].set(1)
        csum = jnp.cumsum(mask)
        n = csum[-1]
        inv = csum[starts] - 1
        rank_idx = jnp.where(mask == 1, csum - 1, N_PATH)
        uniq = jnp.zeros((N_PATH,), jnp.int32).at[rank_idx].set(
            jnp.arange(N_PATH, dtype=jnp.int32), mode='drop')
        pvs += [ends, uniq]
        pes += [inv + eoff, jnp.arange(N_PATH, dtype=jnp.int32) + eoff]
        vals += [jnp.ones((E_PER,), bool), jnp.arange(N_PATH) < n]
        eoff = eoff + n
    pv = jnp.concatenate(pvs)
    pe = jnp.concatenate(pes)
    val = jnp.concatenate(vals)
    pad = jnp.full((_KLEN - _NPAIR,), _BIG, jnp.int32)
    key_e = jnp.concatenate([jnp.where(val, pe * 16384 + pv, _BIG), pad])
    key_v = jnp.concatenate([jnp.where(val, pv * 32768 + pe, _BIG), pad])
    ks_e = jnp.sort(key_e)
    ks_v = jnp.sort(key_v)
    b_e = jnp.searchsorted(
        ks_e, jnp.arange(_NEW + 1, dtype=jnp.int32) * (_EWIN * 16384),
    ).astype(jnp.int32)
    b_v = jnp.searchsorted(
        ks_v, jnp.arange(_NVW + 1, dtype=jnp.int32) * (_VWIN * 32768),
    ).astype(jnp.int32)
    dup_e = jnp.concatenate([jnp.array([False]), ks_e[1:] == ks_e[:-1]])
    dup_v = jnp.concatenate([jnp.array([False]), ks_v[1:] == ks_v[:-1]])
    ks_e = jnp.where(dup_e, _BIG, ks_e)
    ks_v = jnp.where(dup_v, _BIG, ks_v)
    b_e = jnp.concatenate([b_e, jnp.zeros((144 - _NEW - 1,), jnp.int32)])
    b_v = jnp.concatenate([b_v, jnp.zeros((80 - _NVW - 1,), jnp.int32)])
    ok_e = (ks_e < _BIG).astype(jnp.float32)
    ok_v = (ks_v < _BIG).astype(jnp.float32)
    deg_e = jnp.maximum(jax.ops.segment_sum(
        ok_e, jnp.minimum(ks_e >> 14, _EROWS - 1), num_segments=_EROWS), 1.0)
    deg_v = jnp.maximum(jax.ops.segment_sum(
        ok_v, jnp.minimum(ks_v >> 15, _VROWS - 1), num_segments=_VROWS), 1.0)
    return ks_e, ks_v, b_e, b_v, deg_e, deg_v


def _sel(vec_ref, i):
    return vec_ref[pl.ds(i, 16)][0]


_SC_MESH = dict(core_axis_name="c", subcore_axis_name="s")


def _seg_mean_kernel(nwin_per, win_rows, shift, vmask, relu):
    def body(key_hbm, b_hbm, table_hbm, deg_hbm, out_hbm,
             bvec, kb, gi, si, rows, dbuf, acc, sem):
        z16 = jnp.zeros((16,), jnp.float32)
        c = lax.axis_index("c")
        s = lax.axis_index("s")
        g = c * 16 + s
        pltpu.sync_copy(b_hbm, bvec)

        for wi in range(nwin_per):
            w = g * nwin_per + wi
            base = w * win_rows
            lo = _sel(bvec, w)
            hi = _sel(bvec, w + 1)
            lo_key = base << shift
            hi_key = (base + win_rows) << shift

            def zero_row(r, _):
                for i in range(16):
                    acc[r, pl.ds(i * 16, 16)] = z16
                return 0

            lax.fori_loop(0, win_rows + 8, zero_row, 0)

            astart = jnp.bitwise_and(lo, jnp.int32(-8))
            nch = (hi - astart + (_CH - 1)) // _CH

            def chunk(n, carry):
                cstart = pl.multiple_of(astart + n * _CH, 8)
                pltpu.sync_copy(key_hbm.at[pl.ds(cstart, _CH)], kb)
                for i in range(8):
                    sl = pl.ds(i * 16, 16)
                    kk = kb[sl]
                    m = (kk >= lo_key) & (kk < hi_key)
                    gi[sl] = jnp.where(m, jnp.bitwise_and(kk, vmask), 0)
                    si[sl] = jnp.where(m, (kk >> shift) - base,
                                       jnp.int32(win_rows))
                pltpu.async_copy(table_hbm.at[gi], rows, sem).wait()

                def pair(p, carry):
                    cur_e = carry[0]
                    regs = carry[1:]
                    ep = si[pl.ds(p, 16)][0]
                    changed = ep != cur_e

                    @pl.when(changed)
                    def _():
                        for cc in range(16):
                            sl = pl.ds(cc * 16, 16)
                            acc[cur_e, sl] = acc[cur_e, sl] + regs[cc]

                    new_regs = tuple(
                        jnp.where(changed, rows[p, pl.ds(cc * 16, 16)],
                                  regs[cc] + rows[p, pl.ds(cc * 16, 16)])
                        for cc in range(16))
                    return (ep,) + new_regs

                return lax.fori_loop(0, _CH, pair, carry)

            zero16 = jnp.zeros((16,), jnp.float32)
            carry0 = (jnp.int32(win_rows),) + (zero16,) * 16
            carry = lax.fori_loop(0, jnp.maximum(nch, 0), chunk, carry0)
            last_e = carry[0]
            for cc in range(16):
                sl = pl.ds(cc * 16, 16)
                acc[last_e, sl] = acc[last_e, sl] + carry[1 + cc]
            pltpu.sync_copy(deg_hbm.at[pl.ds(base, win_rows)],
                            dbuf.at[pl.ds(0, win_rows)])

            def wb(j, _):
                d = dbuf[pl.ds(j, 16)][0]
                for i in range(16):
                    sl = pl.ds(i * 16, 16)
                    v = acc[j, sl] / d
                    if relu:
                        v = jnp.maximum(v, 0.0)
                    acc[j, sl] = v
                return 0

            lax.fori_loop(0, win_rows, wb, 0)
            pltpu.sync_copy(acc.at[pl.ds(0, win_rows)],
                            out_hbm.at[pl.ds(base, win_rows)])

    return body


def _seg_mean(key_s, bounds, table, deg, nwin_per, win_rows, nrows_out,
              shift, vmask, relu, blen):
    kfn = _seg_mean_kernel(nwin_per, win_rows, shift, vmask, relu)

    @functools.partial(
        pl.kernel, mesh=plsc.VectorSubcoreMesh(**_SC_MESH),
        out_type=jax.ShapeDtypeStruct((nrows_out, 256), jnp.float32),
        scratch_types=[
            pltpu.VMEM((blen,), jnp.int32),
            pltpu.VMEM((_CH,), jnp.int32),
            pltpu.VMEM((_CH,), jnp.int32),
            pltpu.VMEM((_CH + 16,), jnp.int32),
            pltpu.VMEM((_CH, 256), jnp.float32),
            pltpu.VMEM((win_rows + 16,), jnp.float32),
            pltpu.VMEM((win_rows + 8, 256), jnp.float32),
            pltpu.SemaphoreType.DMA,
        ],
    )
    def k(key_hbm, b_hbm, table_hbm, deg_hbm, out_hbm,
          bvec, kb, gi, si, rows, dbuf, acc, sem):
        kfn(key_hbm, b_hbm, table_hbm, deg_hbm, out_hbm,
            bvec, kb, gi, si, rows, dbuf, acc, sem)

    return k(key_s, bounds, table, deg)


def _hop1(x, ks_e, b_e, deg_e):
    return _seg_mean(ks_e, b_e, x, deg_e, 4, _EWIN, _EROWS, 14,
                     jnp.int32(16383), False, 144)


def _hop2(xe, ks_v, b_v, deg_v):
    return _seg_mean(ks_v, b_v, xe, deg_v, 2, _VWIN, _VROWS, 15,
                     jnp.int32(32767), True, 80)


def _linear_kernel(x_ref, w_ref, b_ref, o_ref):
    o_ref[...] = jnp.dot(x_ref[...], w_ref[...],
                         preferred_element_type=jnp.float32) + b_ref[...]


def _linear(x, w_t, b):
    blk = 2000
    return pl.pallas_call(
        _linear_kernel,
        grid=(N_PATH // blk,),
        in_specs=[
            pl.BlockSpec((blk, 256), lambda i: (i, 0)),
            pl.BlockSpec((256, 256), lambda i: (0, 0)),
            pl.BlockSpec((1, 256), lambda i: (0, 0)),
        ],
        out_specs=pl.BlockSpec((blk, 256), lambda i: (i, 0)),
        out_shape=jax.ShapeDtypeStruct((N_PATH, 256), jnp.float32),
    )(x, w_t, b[None])


def _topk_kernel(gen_ref, path_ref, o_ref):
    attn = jnp.dot(gen_ref[...], path_ref[...].T,
                   preferred_element_type=jnp.float32)
    idxmat = jax.lax.broadcasted_iota(jnp.int32, (8, N_PATH), 1)
    for k in range(K_TOP):
        m = jnp.max(attn, axis=1, keepdims=True)
        cand = jnp.where(attn == m, idxmat, jnp.int32(2**30))
        idx = jnp.min(cand, axis=1, keepdims=True)
        o_ref[:, pl.ds(k, 1)] = idx
        attn = jnp.where(idxmat == idx, -jnp.inf, attn)


def _topk(gen, path):
    gen_pad = jnp.zeros((8, 256), jnp.float32).at[:G_TOTAL].set(gen)
    out = pl.pallas_call(
        _topk_kernel,
        out_shape=jax.ShapeDtypeStruct((8, K_TOP), jnp.int32),
    )(gen_pad, path)
    return out[:G_TOTAL]


def _token_kernel(gen_ref, path_ref, cv_ref, ce_ref, ti_ref,
                  ggw_ref, ggb_ref, w1_ref, b1_ref, w2_ref, b2_ref,
                  lng_ref, lnb_ref, mmw_ref, mmb_ref, clsw_ref, clsb_ref,
                  o_ref, t_ref):
    t_ref[...] = jnp.zeros((128, 256), jnp.float32)
    t_ref[0:8, :] = gen_ref[...]

    def body(k, _):
        idx = ti_ref[k // K_TOP, k % K_TOP]
        t_ref[pl.ds(k + G_TOTAL, 1), :] = path_ref[pl.ds(idx, 1), :]
        return 0

    jax.lax.fori_loop(0, 96, body, 0)

    cv = cv_ref[...]
    ce = ce_ref[...]
    C = (cv == cv.T).astype(jnp.float32)
    deg = jnp.sum(C, axis=1, keepdims=True)
    e_row = jax.lax.broadcasted_iota(jnp.int32, (8, 128), 0)
    E_mat = jnp.where(ce == e_row, 1.0 / 17.0, 0.0)
    e_col = jax.lax.broadcasted_iota(jnp.int32, (128, 8), 1)
    oh_ke = (ce.T == e_col).astype(jnp.float32)
    G_mat = jnp.dot(C, oh_ke, preferred_element_type=jnp.float32) / deg

    T = t_ref[...]
    for j in range(3):
        T_lin = jnp.dot(T, ggw_ref[j], preferred_element_type=jnp.float32) \
            + ggb_ref[j][None, :]
        Xe = jnp.dot(E_mat, T_lin, preferred_element_type=jnp.float32)
        T = jnp.maximum(jnp.dot(G_mat, Xe, preferred_element_type=jnp.float32),
                        0.0)

    w1 = w1_ref[...]
    b1 = b1_ref[...]
    w2 = w2_ref[...]
    b2 = b2_ref[...]
    lng = lng_ref[...]
    lnb = lnb_ref[...]

    def gelu_exact(z):
        return 0.5 * z * (1.0 + jax.lax.erf(z * (2.0 ** -0.5)))

    def ffn_ln(x):
        h = gelu_exact(jnp.dot(x, w1, preferred_element_type=jnp.float32)
                       + b1)
        f = jnp.dot(h, w2, preferred_element_type=jnp.float32) + b2
        mu = jnp.mean(f, axis=-1, keepdims=True)
        var = jnp.mean((f - mu) ** 2, axis=-1, keepdims=True)
        return (f - mu) / jnp.sqrt(var + 1e-5) * lng + lnb

    F = ffn_ln(T)
    tc_zero = ffn_ln(jnp.zeros((8, 256), jnp.float32))[0:1]

    r_col = jax.lax.broadcasted_iota(jnp.int32, (1, 128), 1)
    m_g = jnp.where(r_col < G_TOTAL, 1.0 / G_TOTAL, 0.0)
    gene_embed = jnp.dot(m_g, F, preferred_element_type=jnp.float32)

    row_i = jax.lax.broadcasted_iota(jnp.int32, (128, 128), 0)
    col_i = jax.lax.broadcasted_iota(jnp.int32, (128, 128), 1)
    L = jnp.where(col_i < row_i, 1.0, 0.0)
    S = jnp.sum(C * L, axis=1, keepdims=True)
    r_row = jax.lax.broadcasted_iota(jnp.int32, (128, 1), 0)
    m_path = jnp.where((S == 0.0) & (r_row >= G_TOTAL) & (r_row < 102),
                       1.0, 0.0)
    n_unique = jnp.sum(m_path)
    path_sum = jnp.dot(m_path.T, F, preferred_element_type=jnp.float32)
    path_embed = (path_sum + (N_PATH - n_unique) * tc_zero) / N_PATH

    fused = jnp.concatenate([gene_embed, path_embed], axis=1)
    fusion = _relu6(jnp.dot(fused, mmw_ref[...],
                            preferred_element_type=jnp.float32) + mmb_ref[...])
    o_ref[...] = jnp.dot(fusion, clsw_ref[...],
                         preferred_element_type=jnp.float32) + clsb_ref[...]


def _token_head(gen, path, top_idx, p):
    cv = jnp.concatenate([
        jnp.arange(G_TOTAL, dtype=jnp.int32),
        top_idx.reshape(-1).astype(jnp.int32) + G_TOTAL,
        jnp.arange(102, 128, dtype=jnp.int32) + 100000,
    ])[:, None]
    ce = jnp.concatenate([
        jnp.arange(G_TOTAL, dtype=jnp.int32),
        jnp.repeat(jnp.arange(G_TOTAL, dtype=jnp.int32), K_TOP),
        jnp.full((26,), G_TOTAL + 1, jnp.int32),
    ])[None, :]
    gen_pad = jnp.zeros((8, 256), jnp.float32).at[:G_TOTAL].set(gen)
    ggw = jnp.stack([p['gg%d_w' % j].T for j in range(3)])
    ggb = jnp.stack([p['gg%d_b' % j] for j in range(3)])
    return pl.pallas_call(
        _token_kernel,
        in_specs=[
            pl.BlockSpec((8, 256), lambda: (0, 0)),
            pl.BlockSpec((N_PATH, 256), lambda: (0, 0)),
            pl.BlockSpec((128, 1), lambda: (0, 0)),
            pl.BlockSpec((1, 128), lambda: (0, 0)),
            pl.BlockSpec(memory_space=pltpu.SMEM),
            pl.BlockSpec((3, 256, 256), lambda: (0, 0, 0)),
            pl.BlockSpec((3, 256), lambda: (0, 0)),
            pl.BlockSpec((256, 512), lambda: (0, 0)),
            pl.BlockSpec((1, 512), lambda: (0, 0)),
            pl.BlockSpec((512, 256), lambda: (0, 0)),
            pl.BlockSpec((1, 256), lambda: (0, 0)),
            pl.BlockSpec((1, 256), lambda: (0, 0)),
            pl.BlockSpec((1, 256), lambda: (0, 0)),
            pl.BlockSpec((512, 128), lambda: (0, 0)),
            pl.BlockSpec((1, 128), lambda: (0, 0)),
            pl.BlockSpec((128, 4), lambda: (0, 0)),
            pl.BlockSpec((1, 4), lambda: (0, 0)),
        ],
        out_specs=pl.BlockSpec((1, 4), lambda: (0, 0)),
        out_shape=jax.ShapeDtypeStruct((1, 4), jnp.float32),
        scratch_shapes=[pltpu.VMEM((128, 256), jnp.float32)],
    )(gen_pad, path, cv, ce, top_idx.astype(jnp.int32),
      ggw, ggb, p['ffn_w1'].T, p['ffn_b1'][None], p['ffn_w2'].T,
      p['ffn_b2'][None], p['ln_g'][None], p['ln_b'][None],
      p['mm_w'].T, p['mm_b'][None], p['cls_w'].T, p['cls_b'][None])


def kernel(x_ff, x_ffpe, x_omic1, x_omic2, x_omic3, x_omic4, x_omic5, x_omic6,
           ff_edge_index, ffpe_edge_index, share_edge, params):
    p = params
    (ks_e, ks_v, b_e, b_v, deg_e, deg_v) = _pair_keys(
        ff_edge_index, ffpe_edge_index, share_edge)

    x_all = jnp.concatenate([x_ff[0], x_ffpe[0]], axis=0)
    w1s = jnp.stack([p['ff_w1'].T, p['fp_w1'].T])
    b1s = jnp.stack([p['ff_b1'], p['fp_b1']])[:, None, :]
    w2s = jnp.stack([p['ff_w2'].T, p['fp_w2'].T])
    b2s = jnp.stack([p['ff_b2'], p['fp_b2']])[:, None, :]
    path = _path_mlp(x_all, w1s, b1s, w2s, b2s)

    for j in range(3):
        xlin = _linear(path, p['gp%d_w' % j].T, p['gp%d_b' % j])
        xe = _hop1(xlin, ks_e, b_e, deg_e)
        xv = _hop2(xe, ks_v, b_v, deg_v)
        path = xv[:N_PATH]

    omics = [x_omic1, x_omic2, x_omic3, x_omic4, x_omic5, x_omic6]
    gens = []
    for i in range(6):
        g = jax.nn.elu(omics[i] @ p['g%d_w1' % i].T + p['g%d_b1' % i])
        g = jax.nn.elu(g @ p['g%d_w2' % i].T + p['g%d_b2' % i])
        gens.append(g)
    gen = jnp.stack(gens)

    top_idx = _topk(gen, path)
    logits = _token_head(gen, path, top_idx, p)
    return (logits, path, gen)
